# Initial kernel scaffold; baseline (speedup 1.0000x reference)
import jax
import jax.numpy as jnp
from jax.experimental import pallas as pl


def kernel(x, target1, W_gat, a_src, a_dst, b_gat, W_gcn, b_gcn, W_fcg1, b_fcg1, W_fcg2, b_fcg2, emb_xt, w_c2, b_c2, W_fc2xt, b_fc2xt, w_c1, b_c1, W_fc1xt, b_fc1xt, W_fc1, b_fc1, W_fc2, b_fc2, W_out, b_out, edge_index, batch, target2):
    raise NotImplementedError("write your pallas kernel here")


# speedup vs baseline: 1.0300x; 1.0300x over previous
import functools

import jax
import jax.numpy as jnp
from jax import lax
from jax.experimental import pallas as pl

N = 10000
E = 160000
B = 128
F0 = 78
H = 10
HF = F0 * H
FP = 80
HFP = H * FP


def _leaky(x):
    return jnp.where(x >= 0, x, 0.2 * x)


def _head_kernel(xc_ref, w1_ref, b1_ref, w2_ref, b2_ref, wo_ref, bo_ref, out_ref):
    h = jnp.maximum(jnp.dot(xc_ref[...], w1_ref[...],
                            preferred_element_type=jnp.float32) + b1_ref[...], 0.0)
    h = jnp.maximum(jnp.dot(h, w2_ref[...],
                            preferred_element_type=jnp.float32) + b2_ref[...], 0.0)
    out_ref[...] = jnp.dot(h, wo_ref[...],
                           preferred_element_type=jnp.float32) + bo_ref[...]


def _head(xc, W_fc1, b_fc1, W_fc2, b_fc2, W_out, b_out):
    return pl.pallas_call(
        _head_kernel,
        out_shape=jax.ShapeDtypeStruct((B, 1), jnp.float32),
    )(xc, W_fc1, b_fc1[None, :], W_fc2, b_fc2[None, :], W_out, b_out[None, :])


def kernel(x, target1, W_gat, a_src, a_dst, b_gat, W_gcn, b_gcn, W_fcg1, b_fcg1,
           W_fcg2, b_fcg2, emb_xt, w_c2, b_c2, W_fc2xt, b_fc2xt, w_c1, b_c1,
           W_fc1xt, b_fc1xt, W_fc1, b_fc1, W_fc2, b_fc2, W_out, b_out,
           edge_index, batch, target2):
    loops = jnp.arange(N, dtype=edge_index.dtype)
    src = jnp.concatenate([edge_index[0], loops])
    dst = jnp.concatenate([edge_index[1], loops])

    xW = (x @ W_gat).reshape(N, H, F0)
    es = (xW * a_src[None]).sum(-1)
    ed = (xW * a_dst[None]).sum(-1)
    e = _leaky(es[src] + ed[dst])
    ee = jnp.exp(e)
    den = jax.ops.segment_sum(ee, dst, num_segments=N)
    alpha = ee / (den[dst] + 1e-16)
    gat = jax.ops.segment_sum(xW[src] * alpha[:, :, None], dst, num_segments=N)
    h1 = jax.nn.relu(gat.reshape(N, HF) + b_gat)

    deg = jax.ops.segment_sum(jnp.ones(src.shape, jnp.float32), dst, num_segments=N)
    dinv = deg ** -0.5
    norm = dinv[src] * dinv[dst]
    xW2 = h1 @ W_gcn
    h2 = jax.nn.relu(
        jax.ops.segment_sum(xW2[src] * norm[:, None], dst, num_segments=N) + b_gcn)

    gmax = jax.ops.segment_max(h2, batch, num_segments=B)
    gmax = jnp.where(jnp.isfinite(gmax), gmax, 0.0)
    gsum = jax.ops.segment_sum(h2, batch, num_segments=B)
    cnt = jax.ops.segment_sum(jnp.ones((N,), jnp.float32), batch, num_segments=B)
    gmean = gsum / jnp.maximum(cnt, 1.0)[:, None]
    g = jnp.concatenate([gmax, gmean], axis=1)
    g = jax.nn.relu(g @ W_fcg1 + b_fcg1)
    g = g @ W_fcg2 + b_fcg2

    emb = emb_xt[target2]
    o2 = lax.conv_general_dilated(emb, w_c2, (1,), "VALID",
                                  dimension_numbers=("NCH", "OIH", "NCH"))
    xt2 = (o2 + b_c2[None, :, None]).reshape(B, 32 * 121) @ W_fc2xt + b_fc2xt
    o1 = lax.conv_general_dilated(target1, w_c1, (1,), "VALID",
                                  dimension_numbers=("NCH", "OIH", "NCH"))
    xt1 = (o1 + b_c1[None, :, None]).reshape(B, 544) @ W_fc1xt + b_fc1xt

    xc = jnp.concatenate([g, xt1, xt2], axis=1)
    return _head(xc, W_fc1, b_fc1, W_fc2, b_fc2, W_out, b_out)


# speedup vs baseline: 5.6513x; 5.4867x over previous
import functools

import jax
import jax.numpy as jnp
import numpy as np
from jax import lax
from jax.experimental import pallas as pl
from jax.experimental.pallas import tpu as pltpu
from jax.experimental.pallas import tpu_sc as plsc

N = 10000
E = 160000
B = 128
F0 = 78
H = 10
FP = 80
CH = 128
NCHK = 7
HFP = NCHK * CH
NP = 10240
DUMMY = 10100
NWORK = 32
EPW = 5376
NBLK = 42
KE = 128
EPAD = NWORK * EPW

TM_ROWS = 4128
TMD = 256

_COLMAP = np.concatenate([np.arange(h * FP, h * FP + F0) for h in range(H)])
_HEADCOL = np.repeat(np.arange(H), F0)
_VREG_HEAD = [[(c * CH + 16 * j) // FP for j in range(8)] for c in range(NCHK)]

_f32 = jnp.float32
_i32 = jnp.int32


def _mesh():
    return plsc.VectorSubcoreMesh(core_axis_name="c", subcore_axis_name="s",
                                  num_cores=2, num_subcores=16)


def _tca_body(x_ref, wg_ref, asm_ref, adm_ref, *outs):
    xwp = jnp.dot(x_ref[...], wg_ref[...], preferred_element_type=_f32)
    for cidx in range(NCHK):
        outs[cidx][...] = xwp[:, cidx * CH:(cidx + 1) * CH]
    outs[NCHK][...] = jnp.dot(xwp, asm_ref[...], preferred_element_type=_f32)
    outs[NCHK + 1][...] = jnp.dot(xwp, adm_ref[...], preferred_element_type=_f32)


def _tca(x_pad, wg_pad, asm, adm):
    blk = 512
    outs = [jax.ShapeDtypeStruct((NP, CH), _f32) for _ in range(NCHK)]
    outs += [jax.ShapeDtypeStruct((NP, 16), _f32)] * 2
    return pl.pallas_call(
        _tca_body,
        grid=(NP // blk,),
        in_specs=[pl.BlockSpec((blk, F0), lambda i: (i, 0)),
                  pl.BlockSpec((F0, HFP), lambda i: (0, 0)),
                  pl.BlockSpec((HFP, 16), lambda i: (0, 0)),
                  pl.BlockSpec((HFP, 16), lambda i: (0, 0))],
        out_specs=[pl.BlockSpec((blk, CH), lambda i: (i, 0))] * NCHK
                  + [pl.BlockSpec((blk, 16), lambda i: (i, 0))] * 2,
        out_shape=outs,
    )(x_pad, wg_pad, asm, adm)


def _scb():
    @functools.partial(
        pl.kernel,
        out_type=jax.ShapeDtypeStruct((2, NP, 16), _f32),
        mesh=_mesh(),
        compiler_params=pltpu.CompilerParams(use_tc_tiling_on_sc=False),
        scratch_types=[pltpu.VMEM((NBLK, KE), _i32),
                       pltpu.VMEM((NBLK, KE), _i32),
                       pltpu.VMEM((KE, 16), _f32),
                       pltpu.VMEM((KE, 16), _f32),
                       pltpu.VMEM((KE, 16), _f32),
                       pltpu.VMEM_SHARED((NP, 16), _f32),
                       pltpu.SemaphoreType.DMA],
    )
    def k(es_hbm, ed_hbm, srcp, dstp, zero16, den_parts,
          srcv, dstv, esr, edr, eeb, acc, sem):
        c = lax.axis_index("c")
        s = lax.axis_index("s")
        w = s * 2 + c
        lane = lax.iota(_i32, 16)
        keep = lane < 10
        one10 = jnp.where(lane == 10, 1.0, 0.0).astype(_f32)

        @pl.when(s == 0)
        def _():
            pltpu.sync_copy(zero16, acc)
        plsc.subcore_barrier()

        pltpu.sync_copy(srcp.at[w], srcv)
        pltpu.sync_copy(dstp.at[w], dstv)

        @pl.loop(0, NBLK)
        def _blk(blk):
            pltpu.async_copy(es_hbm.at[srcv.at[blk]], esr, sem).wait()
            pltpu.async_copy(ed_hbm.at[dstv.at[blk]], edr, sem).wait()

            @pl.loop(0, KE)
            def _edge(i):
                v = esr[i, :] + edr[i, :]
                v = jnp.maximum(v, 0.0) + 0.2 * jnp.minimum(v, 0.0)
                ee = jnp.exp(v)
                eeb[i, :] = jnp.where(keep, ee, one10)

            pltpu.sync_copy(eeb, acc.at[dstv.at[blk]], add=True)

        plsc.subcore_barrier()
        nr = NP // 16
        pltpu.sync_copy(acc.at[pl.ds(s * nr, nr)],
                        den_parts.at[c, pl.ds(s * nr, nr)])
    return k


def _tcc_body(dp_ref, es_ref, ed_ref, out_ref):
    d = dp_ref[0] + dp_ref[1]
    deg = d[:, 10:11]
    dinv = jnp.where(deg > 0, lax.rsqrt(jnp.maximum(deg, 1e-30)), 0.0)
    col = lax.broadcasted_iota(_i32, (NP, 16), 1)
    denb = jnp.where(col == 10, dinv, d + 1e-16)
    out_ref[...] = jnp.concatenate([es_ref[...], ed_ref[...], denb], axis=1)


def _tcc(den_parts, es16, ed16):
    return pl.pallas_call(
        _tcc_body,
        out_shape=jax.ShapeDtypeStruct((NP, 48), _f32),
    )(den_parts, es16, ed16)


def _scb2():
    @functools.partial(
        pl.kernel,
        out_type=jax.ShapeDtypeStruct((NWORK, EPW, 16), _f32),
        mesh=_mesh(),
        compiler_params=pltpu.CompilerParams(use_tc_tiling_on_sc=False),
        scratch_types=[pltpu.VMEM((NBLK, KE), _i32),
                       pltpu.VMEM((NBLK, KE), _i32),
                       pltpu.VMEM((KE, 48), _f32),
                       pltpu.VMEM((KE, 48), _f32),
                       pltpu.VMEM((KE, 16), _f32),
                       pltpu.SemaphoreType.DMA],
    )
    def k(ct_hbm, srcp, dstp, an_out, srcv, dstv, rs, rd, anb, sem):
        c = lax.axis_index("c")
        s = lax.axis_index("s")
        w = s * 2 + c
        lane = lax.iota(_i32, 16)
        keep = lane < 10
        one10 = jnp.where(lane == 10, 1.0, 0.0).astype(_f32)
        is10 = lane == 10

        pltpu.sync_copy(srcp.at[w], srcv)
        pltpu.sync_copy(dstp.at[w], dstv)

        @pl.loop(0, NBLK)
        def _blk(blk):
            pltpu.async_copy(ct_hbm.at[srcv.at[blk]], rs, sem).wait()
            pltpu.async_copy(ct_hbm.at[dstv.at[blk]], rd, sem).wait()

            @pl.loop(0, KE)
            def _edge(i):
                v = rs[i, pl.ds(0, 16)] + rd[i, pl.ds(16, 16)]
                v = jnp.maximum(v, 0.0) + 0.2 * jnp.minimum(v, 0.0)
                ee = jnp.where(keep, jnp.exp(v), one10)
                dfs = rs[i, pl.ds(32, 16)]
                dfd = rd[i, pl.ds(32, 16)]
                a = jnp.where(keep, ee / dfd, 0.0)
                anb[i, :] = jnp.where(is10, dfs * dfd, a)

            pltpu.sync_copy(anb, an_out.at[w, pl.ds(blk * KE, KE)])
    return k


def _sc_aggregate(gat):
    @functools.partial(
        pl.kernel,
        out_type=jax.ShapeDtypeStruct((NCHK, 2, NP, CH), _f32),
        mesh=_mesh(),
        compiler_params=pltpu.CompilerParams(use_tc_tiling_on_sc=False),
        scratch_types=[pltpu.VMEM((NBLK, KE), _i32),
                       pltpu.VMEM((NBLK, KE), _i32),
                       pltpu.VMEM((KE, 16), _f32),
                       pltpu.VMEM((KE, CH), _f32),
                       pltpu.VMEM_SHARED((NP, CH), _f32),
                       pltpu.SemaphoreType.DMA],
    )
    def k(xw0, xw1, xw2, xw3, xw4, xw5, xw6, srcp, dstp, an_hbm, zero128, out,
          srcv, dstv, anb, rows, acc, sem):
        c = lax.axis_index("c")
        s = lax.axis_index("s")
        w = s * 2 + c

        pltpu.sync_copy(srcp.at[w], srcv)
        pltpu.sync_copy(dstp.at[w], dstv)

        for cidx, xw in enumerate((xw0, xw1, xw2, xw3, xw4, xw5, xw6)):
            heads = _VREG_HEAD[cidx]

            @pl.when(s == 0)
            def _():
                pltpu.sync_copy(zero128, acc)
            plsc.subcore_barrier()

            @pl.loop(0, NBLK)
            def _blk(blk):
                pltpu.sync_copy(an_hbm.at[w, pl.ds(blk * KE, KE)], anb)
                pltpu.async_copy(xw.at[srcv.at[blk]], rows, sem).wait()

                @pl.loop(0, KE)
                def _edge(i):
                    av = anb[i, :]
                    if gat:
                        hs = sorted(set(heads))
                        sv = {h: jnp.full((16,), av[h], _f32) for h in hs}
                        scale = [sv[h] for h in heads]
                    else:
                        v0 = jnp.full((16,), av[10], _f32)
                        scale = [v0] * 8
                    for j in range(8):
                        rows[i, pl.ds(16 * j, 16)] = (
                            rows[i, pl.ds(16 * j, 16)] * scale[j])

                pltpu.sync_copy(rows, acc.at[dstv.at[blk]], add=True)

            plsc.subcore_barrier()
            nr = NP // 16
            pltpu.sync_copy(acc.at[pl.ds(s * nr, nr)],
                            out.at[cidx, c, pl.ds(s * nr, nr)])
            plsc.subcore_barrier()
    return k


def _tce_body(gp_ref, bg_ref, wg_ref, *outs):
    parts = [jnp.maximum(gp_ref[c, 0] + gp_ref[c, 1]
                         + bg_ref[0, c * CH:(c + 1) * CH], 0.0)
             for c in range(NCHK)]
    h1 = jnp.concatenate(parts, axis=1)
    xw2 = jnp.dot(h1, wg_ref[...], preferred_element_type=_f32)
    for cidx in range(NCHK):
        outs[cidx][...] = xw2[:, cidx * CH:(cidx + 1) * CH]


def _tce(gat_parts, bg_pad, wgcn_pad):
    blk = 512
    return pl.pallas_call(
        _tce_body,
        grid=(NP // blk,),
        in_specs=[pl.BlockSpec((NCHK, 2, blk, CH), lambda i: (0, 0, i, 0)),
                  pl.BlockSpec((1, HFP), lambda i: (0, 0)),
                  pl.BlockSpec((HFP, HFP), lambda i: (0, 0))],
        out_specs=[pl.BlockSpec((blk, CH), lambda i: (i, 0))] * NCHK,
        out_shape=[jax.ShapeDtypeStruct((NP, CH), _f32) for _ in range(NCHK)],
    )(gat_parts, bg_pad, wgcn_pad)


def _tcg1_body(gp_ref, bg_ref, *outs):
    for cidx in range(NCHK):
        outs[cidx][...] = jnp.maximum(
            gp_ref[cidx, 0] + gp_ref[cidx, 1]
            + bg_ref[0, cidx * CH:(cidx + 1) * CH], 0.0)


def _tcg1(gcn_parts, bg_pad):
    blk = 512
    return pl.pallas_call(
        _tcg1_body,
        grid=(NP // blk,),
        in_specs=[pl.BlockSpec((NCHK, 2, blk, CH), lambda i: (0, 0, i, 0)),
                  pl.BlockSpec((1, HFP), lambda i: (0, 0))],
        out_specs=[pl.BlockSpec((blk, CH), lambda i: (i, 0))] * NCHK,
        out_shape=[jax.ShapeDtypeStruct((NP, CH), _f32) for _ in range(NCHK)],
    )(gcn_parts, bg_pad)


def _scg():
    rpw = NP // NWORK
    @functools.partial(
        pl.kernel,
        out_type=[jax.ShapeDtypeStruct((NCHK, NWORK, B, CH), _f32),
                  jax.ShapeDtypeStruct((NCHK, NWORK, B, CH), _f32)],
        mesh=_mesh(),
        compiler_params=pltpu.CompilerParams(use_tc_tiling_on_sc=False),
        scratch_types=[pltpu.VMEM((rpw,), _i32),
                       pltpu.VMEM((64, CH), _f32),
                       pltpu.VMEM((B, CH), _f32),
                       pltpu.VMEM((B, CH), _f32),
                       pltpu.SemaphoreType.DMA],
    )
    def k(h0, h1, h2, h3, h4, h5, h6, batch_hbm, zero128, gmaxp, gsump,
          bv, rowb, pmax, psum, sem):
        c = lax.axis_index("c")
        s = lax.axis_index("s")
        w = s * 2 + c
        one15 = jnp.where(lax.iota(_i32, 16) == 15, 1.0, 0.0).astype(_f32)

        pltpu.sync_copy(batch_hbm.at[pl.ds(w * rpw, rpw)], bv)

        for cidx, h2c in enumerate((h0, h1, h2, h3, h4, h5, h6)):
            pltpu.sync_copy(zero128.at[pl.ds(0, B)], pmax)
            pltpu.sync_copy(zero128.at[pl.ds(0, B)], psum)

            @pl.loop(0, rpw // 64)
            def _sb(sb):
                pltpu.sync_copy(h2c.at[pl.ds(w * rpw + sb * 64, 64)], rowb)

                @pl.loop(0, 4)
                def _grp(gi):
                    idv = bv[pl.ds(sb * 64 + gi * 16, 16)]
                    for jj in range(16):
                        g = idv[jj]
                        i = gi * 16 + jj

                        @pl.when(g < B)
                        def _():
                            for j in range(8):
                                r = rowb[i, pl.ds(16 * j, 16)]
                                pmax[g, pl.ds(16 * j, 16)] = jnp.maximum(
                                    pmax[g, pl.ds(16 * j, 16)], r)
                                psum[g, pl.ds(16 * j, 16)] = (
                                    psum[g, pl.ds(16 * j, 16)] + r)
                            if cidx == NCHK - 1:
                                psum[g, pl.ds(112, 16)] = (
                                    psum[g, pl.ds(112, 16)] + one15)

            pltpu.sync_copy(pmax, gmaxp.at[cidx, w])
            pltpu.sync_copy(psum, gsump.at[cidx, w])
    return k


def _sch2():
    nblk = 32
    @functools.partial(
        pl.kernel,
        out_type=jax.ShapeDtypeStruct((2, 2, TM_ROWS, CH), _f32),
        mesh=_mesh(),
        compiler_params=pltpu.CompilerParams(use_tc_tiling_on_sc=False),
        scratch_types=[pltpu.VMEM((nblk, KE), _i32),
                       pltpu.VMEM((nblk, KE), _i32),
                       pltpu.VMEM((KE, CH), _f32),
                       pltpu.VMEM((KE, CH), _f32),
                       pltpu.VMEM_SHARED((TM_ROWS, CH), _f32),
                       pltpu.VMEM_SHARED((TM_ROWS, CH), _f32),
                       pltpu.SemaphoreType.DMA],
    )
    def k(w2da, w2db, widx, t2idx, zero128, tm_parts,
          wiv, tiv, rowsa, rowsb, acca, accb, sem):
        c = lax.axis_index("c")
        s = lax.axis_index("s")
        w = s * 2 + c

        @pl.when(s == 0)
        def _():
            pltpu.sync_copy(zero128.at[pl.ds(0, TM_ROWS)], acca)
            pltpu.sync_copy(zero128.at[pl.ds(0, TM_ROWS)], accb)
        plsc.subcore_barrier()

        pltpu.sync_copy(widx, wiv)
        pltpu.sync_copy(t2idx.at[w], tiv)

        @pl.loop(0, nblk)
        def _blk(blk):
            pltpu.async_copy(w2da.at[wiv.at[blk]], rowsa, sem).wait()
            pltpu.sync_copy(rowsa, acca.at[tiv.at[blk]], add=True)
            pltpu.async_copy(w2db.at[wiv.at[blk]], rowsb, sem).wait()
            pltpu.sync_copy(rowsb, accb.at[tiv.at[blk]], add=True)

        plsc.subcore_barrier()
        nr = TM_ROWS // 16
        pltpu.sync_copy(acca.at[pl.ds(s * nr, nr)],
                        tm_parts.at[c, 0, pl.ds(s * nr, nr)])
        pltpu.sync_copy(accb.at[pl.ds(s * nr, nr)],
                        tm_parts.at[c, 1, pl.ds(s * nr, nr)])
    return k


def _sct():
    bpw = B // NWORK
    @functools.partial(
        pl.kernel,
        out_type=jax.ShapeDtypeStruct((2, B * TMD, 32), _f32),
        mesh=_mesh(),
        compiler_params=pltpu.CompilerParams(use_tc_tiling_on_sc=False,
                                             needs_layout_passes=False),
        scratch_types=[pltpu.VMEM((32, CH), _f32),
                       pltpu.VMEM((32, CH), _f32),
                       pltpu.VMEM((TMD, 32), _f32),
                       pltpu.SemaphoreType.DMA],
    )
    def k(tm_parts, tmf_out, bufa, bufb, tbuf, sem):
        c = lax.axis_index("c")
        s = lax.axis_index("s")
        w = s * 2 + c

        for cc in range(2):
            @pl.loop(0, bpw)
            def _b(bi):
                b = w * bpw + bi
                pltpu.sync_copy(tm_parts.at[cc, 0, pl.ds(b * 32, 32)], bufa)
                pltpu.sync_copy(tm_parts.at[cc, 1, pl.ds(b * 32, 32)], bufb)

                @pl.loop(0, CH)
                def _j(j):
                    jv = jnp.full((16,), j, _i32)
                    ri0 = lax.iota(_i32, 16)
                    ri1 = ri0 + 16
                    tbuf[j, pl.ds(0, 16)] = plsc.load_gather(bufa, [ri0, jv])
                    tbuf[j, pl.ds(16, 16)] = plsc.load_gather(bufa, [ri1, jv])
                    tbuf[CH + j, pl.ds(0, 16)] = plsc.load_gather(bufb, [ri0, jv])
                    tbuf[CH + j, pl.ds(16, 16)] = plsc.load_gather(bufb, [ri1, jv])

                pltpu.sync_copy(tbuf, tmf_out.at[cc, pl.ds(b * TMD, TMD)])
    return k


def _tcu_body(tmf_ref, e_ref, u_ref):
    u_ref[...] = jnp.dot(tmf_ref[0] + tmf_ref[1], e_ref[...],
                         preferred_element_type=_f32)


def _tcu(tmf, e32):
    blk = 2048
    return pl.pallas_call(
        _tcu_body,
        grid=(B * TMD // blk,),
        in_specs=[pl.BlockSpec((2, blk, 32), lambda i: (0, i, 0)),
                  pl.BlockSpec((32, 128), lambda i: (0, 0))],
        out_specs=pl.BlockSpec((blk, 128), lambda i: (i, 0)),
        out_shape=jax.ShapeDtypeStruct((B * TMD, 128), _f32),
    )(tmf, e32)


def _tch_body(gmaxp, gsump, x3, w1f, bc1, wx1, bx1, out2f, wx2, bx2, bc2rep,
              wfcg1, bfcg1, wfcg2, bfcg2, wfc1, bfc1, wfc2, bfc2, wout, bout,
              out_ref):
    sums = [jnp.sum(gsump[cidx], axis=0) for cidx in range(NCHK)]
    cnt = sums[NCHK - 1][:, CH - 1:CH]
    inv = 1.0 / jnp.maximum(cnt, 1.0)
    parts = [jnp.max(gmaxp[cidx], axis=0) for cidx in range(NCHK)]
    parts += [sm * inv for sm in sums]
    g = jnp.concatenate(parts, axis=1)
    g = jnp.maximum(jnp.dot(g, wfcg1[...], preferred_element_type=_f32)
                    + bfcg1[...], 0.0)
    g = jnp.dot(g, wfcg2[...], preferred_element_type=_f32) + bfcg2[...]

    xt1 = bx1[...]
    for p in range(17):
        o1p = jnp.dot(x3[p], w1f[...], preferred_element_type=_f32) + bc1[...]
        xt1 = xt1 + jnp.dot(o1p, wx1[p * 32:(p + 1) * 32, :],
                            preferred_element_type=_f32)

    xt2 = (jnp.dot(out2f[...], wx2[...], preferred_element_type=_f32)
           + jnp.dot(bc2rep[...], wx2[...], preferred_element_type=_f32)
           + bx2[...])

    xc = jnp.concatenate([g, xt1, xt2], axis=1)
    h = jnp.maximum(jnp.dot(xc, wfc1[...], preferred_element_type=_f32)
                    + bfc1[...], 0.0)
    h = jnp.maximum(jnp.dot(h, wfc2[...], preferred_element_type=_f32)
                    + bfc2[...], 0.0)
    out_ref[...] = jnp.dot(h, wout[...], preferred_element_type=_f32) + bout[...]


def _tch(*args):
    return pl.pallas_call(
        _tch_body,
        out_shape=jax.ShapeDtypeStruct((B, 1), _f32),
    )(*args)


def kernel(x, target1, W_gat, a_src, a_dst, b_gat, W_gcn, b_gcn, W_fcg1, b_fcg1,
           W_fcg2, b_fcg2, emb_xt, w_c2, b_c2, W_fc2xt, b_fc2xt, w_c1, b_c1,
           W_fc1xt, b_fc1xt, W_fc1, b_fc1, W_fc2, b_fc2, W_out, b_out,
           edge_index, batch, target2):
    x_pad = jnp.pad(x, ((0, NP - N), (0, 0)))
    wg_pad = jnp.zeros((F0, HFP), _f32).at[:, _COLMAP].set(W_gat)
    asm = jnp.zeros((HFP, 16), _f32).at[_COLMAP, _HEADCOL].set(a_src.reshape(-1))
    adm = jnp.zeros((HFP, 16), _f32).at[_COLMAP, _HEADCOL].set(a_dst.reshape(-1))
    bg_pad = jnp.zeros((1, HFP), _f32).at[0, _COLMAP].set(b_gat)
    wgcn_pad = (jnp.zeros((HFP, HFP), _f32)
                .at[np.ix_(_COLMAP, _COLMAP)].set(W_gcn))
    bgcn_pad = jnp.zeros((1, HFP), _f32).at[0, _COLMAP].set(b_gcn)
    wfcg1_pad = (jnp.zeros((2 * HFP, 1500), _f32)
                 .at[np.concatenate([_COLMAP, HFP + _COLMAP])].set(W_fcg1))

    loops = jnp.arange(N, dtype=_i32)
    srcp = jnp.concatenate(
        [edge_index[0].astype(_i32), loops,
         jnp.zeros((EPAD - N - E,), _i32)]).reshape(NWORK, NBLK, KE)
    dstp = jnp.concatenate(
        [edge_index[1].astype(_i32), loops,
         jnp.full((EPAD - N - E,), DUMMY, _i32)]).reshape(NWORK, NBLK, KE)
    batch_pad = jnp.concatenate(
        [batch.astype(_i32), jnp.full((NP - N,), B + 2, _i32)])

    zero16 = jnp.zeros((NP, 16), _f32)
    zero128 = jnp.zeros((NP, CH), _f32)

    t2idx = (jnp.arange(B, dtype=_i32)[:, None] * 32 + target2.astype(_i32))
    t2idx = jnp.pad(t2idx.reshape(NWORK, 4000), ((0, 0), (0, 96)),
                    constant_values=4096).reshape(NWORK, 32, KE)
    widx = jnp.pad(jnp.tile(jnp.arange(1000, dtype=_i32), 4),
                   (0, 96)).reshape(32, KE)
    w2d = jnp.transpose(w_c2, (2, 0, 1)).reshape(TMD, 1000).T
    w2da, w2db = w2d[:, :CH], w2d[:, CH:]
    e32 = jnp.pad(emb_xt, ((0, 6), (0, 0)))

    x3 = jnp.stack([target1[:, :, p:p + 8].reshape(B, 160) for p in range(17)])
    wx1 = W_fc1xt.reshape(32, 17, 128).transpose(1, 0, 2).reshape(544, 128)
    w1f = jnp.transpose(w_c1, (1, 2, 0)).reshape(160, 32)
    bc2rep = jnp.repeat(b_c2, 121)[None, :]

    outs = _tca(x_pad, wg_pad, asm, adm)
    xwc, es16, ed16 = outs[:NCHK], outs[NCHK], outs[NCHK + 1]
    den_parts = _scb()(es16, ed16, srcp, dstp, zero16)
    comb = _tcc(den_parts, es16, ed16)
    an = _scb2()(comb, srcp, dstp)
    gat_parts = _sc_aggregate(True)(*xwc, srcp, dstp, an, zero128)
    xw2c = _tce(gat_parts, bg_pad, wgcn_pad)
    gcn_parts = _sc_aggregate(False)(*xw2c, srcp, dstp, an, zero128)
    h2c = _tcg1(gcn_parts, bgcn_pad)
    gmaxp, gsump = _scg()(*h2c, batch_pad, zero128)

    tm_parts = _sch2()(w2da, w2db, widx, t2idx, zero128)
    tmf = _sct()(tm_parts)
    u = _tcu(tmf, e32)

    u3 = u.reshape(B, TMD, 128)
    out2 = sum(u3[:, 32 * kk:32 * kk + 32, kk:kk + 121] for kk in range(8))
    out2f = out2.reshape(B, 32 * 121)

    return _tch(gmaxp, gsump, x3, w1f, b_c1[None, :], wx1, b_fc1xt[None, :],
                out2f, W_fc2xt, b_fc2xt[None, :], bc2rep,
                wfcg1_pad, b_fcg1[None, :], W_fcg2, b_fcg2[None, :],
                W_fc1, b_fc1[None, :], W_fc2, b_fc2[None, :],
                W_out, b_out[None, :])


# speedup vs baseline: 6.2684x; 1.1092x over previous
import functools

import jax
import jax.numpy as jnp
import numpy as np
from jax import lax
from jax.experimental import pallas as pl
from jax.experimental.pallas import tpu as pltpu
from jax.experimental.pallas import tpu_sc as plsc

N = 10000
E = 160000
B = 128
F0 = 78
H = 10
FP = 80
CH = 128
NCHK = 7
HFP = NCHK * CH
NP = 10240
DUMMY = 10100
NWORK = 32
EPW = 5376
NBLK = 42
KE = 128
EPAD = NWORK * EPW

TM_ROWS = 4128
TMD = 256

_COLMAP = np.concatenate([np.arange(h * FP, h * FP + F0) for h in range(H)])
_HEADCOL = np.repeat(np.arange(H), F0)
_VREG_HEAD = [[(c * CH + 16 * j) // FP for j in range(8)] for c in range(NCHK)]

_f32 = jnp.float32
_i32 = jnp.int32


def _mesh():
    return plsc.VectorSubcoreMesh(core_axis_name="c", subcore_axis_name="s",
                                  num_cores=2, num_subcores=16)


def _tca_body(x_ref, wg_ref, asm_ref, adm_ref, *outs):
    xwp = jnp.dot(x_ref[...], wg_ref[...], preferred_element_type=_f32)
    for cidx in range(NCHK):
        outs[cidx][...] = xwp[:, cidx * CH:(cidx + 1) * CH]
    outs[NCHK][...] = jnp.dot(xwp, asm_ref[...], preferred_element_type=_f32)
    outs[NCHK + 1][...] = jnp.dot(xwp, adm_ref[...], preferred_element_type=_f32)


def _tca(x_pad, wg_pad, asm, adm):
    blk = 512
    outs = [jax.ShapeDtypeStruct((NP, CH), _f32) for _ in range(NCHK)]
    outs += [jax.ShapeDtypeStruct((NP, 16), _f32)] * 2
    return pl.pallas_call(
        _tca_body,
        grid=(NP // blk,),
        in_specs=[pl.BlockSpec((blk, F0), lambda i: (i, 0)),
                  pl.BlockSpec((F0, HFP), lambda i: (0, 0)),
                  pl.BlockSpec((HFP, 16), lambda i: (0, 0)),
                  pl.BlockSpec((HFP, 16), lambda i: (0, 0))],
        out_specs=[pl.BlockSpec((blk, CH), lambda i: (i, 0))] * NCHK
                  + [pl.BlockSpec((blk, 16), lambda i: (i, 0))] * 2,
        out_shape=outs,
    )(x_pad, wg_pad, asm, adm)


def _scb():
    @functools.partial(
        pl.kernel,
        out_type=jax.ShapeDtypeStruct((2, NP, 16), _f32),
        mesh=_mesh(),
        compiler_params=pltpu.CompilerParams(use_tc_tiling_on_sc=False),
        scratch_types=[pltpu.VMEM((NBLK, KE), _i32),
                       pltpu.VMEM((NBLK, KE), _i32),
                       pltpu.VMEM((KE, 16), _f32),
                       pltpu.VMEM((KE, 16), _f32),
                       pltpu.VMEM((KE, 16), _f32),
                       pltpu.VMEM_SHARED((NP, 16), _f32),
                       pltpu.SemaphoreType.DMA],
    )
    def k(es_hbm, ed_hbm, srcp, dstp, zero16, den_parts,
          srcv, dstv, esr, edr, eeb, acc, sem):
        c = lax.axis_index("c")
        s = lax.axis_index("s")
        w = s * 2 + c
        lane = lax.iota(_i32, 16)
        keep = lane < 10
        one10 = jnp.where(lane == 10, 1.0, 0.0).astype(_f32)

        @pl.when(s == 0)
        def _():
            pltpu.sync_copy(zero16, acc)
        plsc.subcore_barrier()

        pltpu.sync_copy(srcp.at[w], srcv)
        pltpu.sync_copy(dstp.at[w], dstv)

        @pl.loop(0, NBLK)
        def _blk(blk):
            pltpu.async_copy(es_hbm.at[srcv.at[blk]], esr, sem).wait()
            pltpu.async_copy(ed_hbm.at[dstv.at[blk]], edr, sem).wait()

            @pl.loop(0, KE)
            def _edge(i):
                v = esr[i, :] + edr[i, :]
                v = jnp.maximum(v, 0.0) + 0.2 * jnp.minimum(v, 0.0)
                ee = jnp.exp(v)
                eeb[i, :] = jnp.where(keep, ee, one10)

            pltpu.sync_copy(eeb, acc.at[dstv.at[blk]], add=True)

        plsc.subcore_barrier()
        nr = NP // 16
        pltpu.sync_copy(acc.at[pl.ds(s * nr, nr)],
                        den_parts.at[c, pl.ds(s * nr, nr)])
    return k


def _tcc_body(dp_ref, es_ref, ed_ref, out_ref):
    d = dp_ref[0] + dp_ref[1]
    deg = d[:, 10:11]
    dinv = jnp.where(deg > 0, lax.rsqrt(jnp.maximum(deg, 1e-30)), 0.0)
    col = lax.broadcasted_iota(_i32, (NP, 16), 1)
    denb = jnp.where(col == 10, dinv, d + 1e-16)
    out_ref[...] = jnp.concatenate([es_ref[...], ed_ref[...], denb], axis=1)


def _tcc(den_parts, es16, ed16):
    return pl.pallas_call(
        _tcc_body,
        out_shape=jax.ShapeDtypeStruct((NP, 48), _f32),
    )(den_parts, es16, ed16)


def _scb2():
    @functools.partial(
        pl.kernel,
        out_type=jax.ShapeDtypeStruct((NWORK, EPW, 16), _f32),
        mesh=_mesh(),
        compiler_params=pltpu.CompilerParams(use_tc_tiling_on_sc=False),
        scratch_types=[pltpu.VMEM((NBLK, KE), _i32),
                       pltpu.VMEM((NBLK, KE), _i32),
                       pltpu.VMEM((KE, 48), _f32),
                       pltpu.VMEM((KE, 48), _f32),
                       pltpu.VMEM((KE, 16), _f32),
                       pltpu.SemaphoreType.DMA],
    )
    def k(ct_hbm, srcp, dstp, an_out, srcv, dstv, rs, rd, anb, sem):
        c = lax.axis_index("c")
        s = lax.axis_index("s")
        w = s * 2 + c
        lane = lax.iota(_i32, 16)
        keep = lane < 10
        one10 = jnp.where(lane == 10, 1.0, 0.0).astype(_f32)
        is10 = lane == 10

        pltpu.sync_copy(srcp.at[w], srcv)
        pltpu.sync_copy(dstp.at[w], dstv)

        @pl.loop(0, NBLK)
        def _blk(blk):
            pltpu.async_copy(ct_hbm.at[srcv.at[blk]], rs, sem).wait()
            pltpu.async_copy(ct_hbm.at[dstv.at[blk]], rd, sem).wait()

            @pl.loop(0, KE)
            def _edge(i):
                v = rs[i, pl.ds(0, 16)] + rd[i, pl.ds(16, 16)]
                v = jnp.maximum(v, 0.0) + 0.2 * jnp.minimum(v, 0.0)
                ee = jnp.where(keep, jnp.exp(v), one10)
                dfs = rs[i, pl.ds(32, 16)]
                dfd = rd[i, pl.ds(32, 16)]
                a = jnp.where(keep, ee / dfd, 0.0)
                anb[i, :] = jnp.where(is10, dfs * dfd, a)

            pltpu.sync_copy(anb, an_out.at[w, pl.ds(blk * KE, KE)])
    return k


def _sc_aggregate(gat):
    @functools.partial(
        pl.kernel,
        out_type=jax.ShapeDtypeStruct((NCHK, 2, NP, CH), _f32),
        mesh=_mesh(),
        compiler_params=pltpu.CompilerParams(use_tc_tiling_on_sc=False),
        scratch_types=[pltpu.VMEM((NBLK, KE), _i32),
                       pltpu.VMEM((NBLK, KE), _i32),
                       pltpu.VMEM((KE, 16), _f32),
                       pltpu.VMEM((KE, 16), _f32),
                       pltpu.VMEM((KE, CH), _f32),
                       pltpu.VMEM((KE, CH), _f32),
                       pltpu.VMEM_SHARED((NP, CH), _f32),
                       pltpu.SemaphoreType.DMA,
                       pltpu.SemaphoreType.DMA],
    )
    def k(xw0, xw1, xw2, xw3, xw4, xw5, xw6, srcp, dstp, an_hbm, zero128, out,
          srcv, dstv, anb0, anb1, rows0, rows1, acc, sem, sem2):
        c = lax.axis_index("c")
        s = lax.axis_index("s")
        w = s * 2 + c

        pltpu.sync_copy(srcp.at[w], srcv)
        pltpu.sync_copy(dstp.at[w], dstv)

        for cidx, xw in enumerate((xw0, xw1, xw2, xw3, xw4, xw5, xw6)):
            heads = _VREG_HEAD[cidx]

            @pl.when(s == 0)
            def _():
                pltpu.sync_copy(zero128, acc)
            plsc.subcore_barrier()

            pltpu.async_copy(xw.at[srcv.at[0]], rows0, sem)
            pltpu.async_copy(an_hbm.at[w, pl.ds(0, KE)], anb0, sem2)

            @pl.loop(0, NBLK // 2)
            def _blk2(b2):
                for ph, (rows, anb, rnxt, anxt) in enumerate(
                        ((rows0, anb0, rows1, anb1),
                         (rows1, anb1, rows0, anb0))):
                    blk = b2 * 2 + ph
                    pltpu.make_async_copy(xw.at[srcv.at[blk]], rows, sem).wait()
                    pltpu.make_async_copy(
                        an_hbm.at[w, pl.ds(blk * KE, KE)], anb, sem2).wait()

                    @pl.when(blk + 1 < NBLK)
                    def _():
                        pltpu.async_copy(xw.at[srcv.at[blk + 1]], rnxt, sem)
                        pltpu.async_copy(
                            an_hbm.at[w, pl.ds((blk + 1) * KE, KE)], anxt, sem2)

                    @pl.loop(0, KE, unroll=4)
                    def _edge(i):
                        av = anb[i, :]
                        if gat:
                            hs = sorted(set(heads))
                            sv = {h: jnp.full((16,), av[h], _f32) for h in hs}
                            scale = [sv[h] for h in heads]
                        else:
                            v0 = jnp.full((16,), av[10], _f32)
                            scale = [v0] * 8
                        for j in range(8):
                            rows[i, pl.ds(16 * j, 16)] = (
                                rows[i, pl.ds(16 * j, 16)] * scale[j])

                    pltpu.sync_copy(rows, acc.at[dstv.at[blk]], add=True)

            plsc.subcore_barrier()
            nr = NP // 16
            pltpu.sync_copy(acc.at[pl.ds(s * nr, nr)],
                            out.at[cidx, c, pl.ds(s * nr, nr)])
            plsc.subcore_barrier()
    return k


def _tce_body(gp_ref, bg_ref, wg_ref, *outs):
    parts = [jnp.maximum(gp_ref[c, 0] + gp_ref[c, 1]
                         + bg_ref[0, c * CH:(c + 1) * CH], 0.0)
             for c in range(NCHK)]
    h1 = jnp.concatenate(parts, axis=1)
    xw2 = jnp.dot(h1, wg_ref[...], preferred_element_type=_f32)
    for cidx in range(NCHK):
        outs[cidx][...] = xw2[:, cidx * CH:(cidx + 1) * CH]


def _tce(gat_parts, bg_pad, wgcn_pad):
    blk = 512
    return pl.pallas_call(
        _tce_body,
        grid=(NP // blk,),
        in_specs=[pl.BlockSpec((NCHK, 2, blk, CH), lambda i: (0, 0, i, 0)),
                  pl.BlockSpec((1, HFP), lambda i: (0, 0)),
                  pl.BlockSpec((HFP, HFP), lambda i: (0, 0))],
        out_specs=[pl.BlockSpec((blk, CH), lambda i: (i, 0))] * NCHK,
        out_shape=[jax.ShapeDtypeStruct((NP, CH), _f32) for _ in range(NCHK)],
    )(gat_parts, bg_pad, wgcn_pad)


def _tcg1_body(gp_ref, bg_ref, *outs):
    for cidx in range(NCHK):
        outs[cidx][...] = jnp.maximum(
            gp_ref[cidx, 0] + gp_ref[cidx, 1]
            + bg_ref[0, cidx * CH:(cidx + 1) * CH], 0.0)


def _tcg1(gcn_parts, bg_pad):
    blk = 512
    return pl.pallas_call(
        _tcg1_body,
        grid=(NP // blk,),
        in_specs=[pl.BlockSpec((NCHK, 2, blk, CH), lambda i: (0, 0, i, 0)),
                  pl.BlockSpec((1, HFP), lambda i: (0, 0))],
        out_specs=[pl.BlockSpec((blk, CH), lambda i: (i, 0))] * NCHK,
        out_shape=[jax.ShapeDtypeStruct((NP, CH), _f32) for _ in range(NCHK)],
    )(gcn_parts, bg_pad)


def _scg():
    rpw = NP // NWORK
    @functools.partial(
        pl.kernel,
        out_type=[jax.ShapeDtypeStruct((NCHK, NWORK, B, CH), _f32),
                  jax.ShapeDtypeStruct((NCHK, NWORK, B, CH), _f32)],
        mesh=_mesh(),
        compiler_params=pltpu.CompilerParams(use_tc_tiling_on_sc=False),
        scratch_types=[pltpu.VMEM((rpw,), _i32),
                       pltpu.VMEM((64, CH), _f32),
                       pltpu.VMEM((B, CH), _f32),
                       pltpu.VMEM((B, CH), _f32),
                       pltpu.SemaphoreType.DMA],
    )
    def k(h0, h1, h2, h3, h4, h5, h6, batch_hbm, zero128, gmaxp, gsump,
          bv, rowb, pmax, psum, sem):
        c = lax.axis_index("c")
        s = lax.axis_index("s")
        w = s * 2 + c
        one15 = jnp.where(lax.iota(_i32, 16) == 15, 1.0, 0.0).astype(_f32)

        pltpu.sync_copy(batch_hbm.at[pl.ds(w * rpw, rpw)], bv)

        for cidx, h2c in enumerate((h0, h1, h2, h3, h4, h5, h6)):
            pltpu.sync_copy(zero128.at[pl.ds(0, B)], pmax)
            pltpu.sync_copy(zero128.at[pl.ds(0, B)], psum)

            @pl.loop(0, rpw // 64)
            def _sb(sb):
                pltpu.sync_copy(h2c.at[pl.ds(w * rpw + sb * 64, 64)], rowb)

                @pl.loop(0, 4)
                def _grp(gi):
                    idv = bv[pl.ds(sb * 64 + gi * 16, 16)]
                    for jj in range(16):
                        g = idv[jj]
                        i = gi * 16 + jj

                        @pl.when(g < B)
                        def _():
                            for j in range(8):
                                r = rowb[i, pl.ds(16 * j, 16)]
                                pmax[g, pl.ds(16 * j, 16)] = jnp.maximum(
                                    pmax[g, pl.ds(16 * j, 16)], r)
                                psum[g, pl.ds(16 * j, 16)] = (
                                    psum[g, pl.ds(16 * j, 16)] + r)
                            if cidx == NCHK - 1:
                                psum[g, pl.ds(112, 16)] = (
                                    psum[g, pl.ds(112, 16)] + one15)

            pltpu.sync_copy(pmax, gmaxp.at[cidx, w])
            pltpu.sync_copy(psum, gsump.at[cidx, w])
    return k


def _sch2():
    nblk = 32
    @functools.partial(
        pl.kernel,
        out_type=jax.ShapeDtypeStruct((2, 2, TM_ROWS, CH), _f32),
        mesh=_mesh(),
        compiler_params=pltpu.CompilerParams(use_tc_tiling_on_sc=False),
        scratch_types=[pltpu.VMEM((nblk, KE), _i32),
                       pltpu.VMEM((nblk, KE), _i32),
                       pltpu.VMEM((KE, CH), _f32),
                       pltpu.VMEM((KE, CH), _f32),
                       pltpu.VMEM_SHARED((TM_ROWS, CH), _f32),
                       pltpu.VMEM_SHARED((TM_ROWS, CH), _f32),
                       pltpu.SemaphoreType.DMA],
    )
    def k(w2da, w2db, widx, t2idx, zero128, tm_parts,
          wiv, tiv, rowsa, rowsb, acca, accb, sem):
        c = lax.axis_index("c")
        s = lax.axis_index("s")
        w = s * 2 + c

        @pl.when(s == 0)
        def _():
            pltpu.sync_copy(zero128.at[pl.ds(0, TM_ROWS)], acca)
            pltpu.sync_copy(zero128.at[pl.ds(0, TM_ROWS)], accb)
        plsc.subcore_barrier()

        pltpu.sync_copy(widx, wiv)
        pltpu.sync_copy(t2idx.at[w], tiv)

        @pl.loop(0, nblk)
        def _blk(blk):
            pltpu.async_copy(w2da.at[wiv.at[blk]], rowsa, sem).wait()
            pltpu.sync_copy(rowsa, acca.at[tiv.at[blk]], add=True)
            pltpu.async_copy(w2db.at[wiv.at[blk]], rowsb, sem).wait()
            pltpu.sync_copy(rowsb, accb.at[tiv.at[blk]], add=True)

        plsc.subcore_barrier()
        nr = TM_ROWS // 16
        pltpu.sync_copy(acca.at[pl.ds(s * nr, nr)],
                        tm_parts.at[c, 0, pl.ds(s * nr, nr)])
        pltpu.sync_copy(accb.at[pl.ds(s * nr, nr)],
                        tm_parts.at[c, 1, pl.ds(s * nr, nr)])
    return k


def _sct():
    bpw = B // NWORK
    @functools.partial(
        pl.kernel,
        out_type=jax.ShapeDtypeStruct((2, B * TMD, 32), _f32),
        mesh=_mesh(),
        compiler_params=pltpu.CompilerParams(use_tc_tiling_on_sc=False,
                                             needs_layout_passes=False),
        scratch_types=[pltpu.VMEM((32, CH), _f32),
                       pltpu.VMEM((32, CH), _f32),
                       pltpu.VMEM((TMD, 32), _f32),
                       pltpu.SemaphoreType.DMA],
    )
    def k(tm_parts, tmf_out, bufa, bufb, tbuf, sem):
        c = lax.axis_index("c")
        s = lax.axis_index("s")
        w = s * 2 + c

        for cc in range(2):
            @pl.loop(0, bpw)
            def _b(bi):
                b = w * bpw + bi
                pltpu.sync_copy(tm_parts.at[cc, 0, pl.ds(b * 32, 32)], bufa)
                pltpu.sync_copy(tm_parts.at[cc, 1, pl.ds(b * 32, 32)], bufb)

                @pl.loop(0, CH)
                def _j(j):
                    jv = jnp.full((16,), j, _i32)
                    ri0 = lax.iota(_i32, 16)
                    ri1 = ri0 + 16
                    tbuf[j, pl.ds(0, 16)] = plsc.load_gather(bufa, [ri0, jv])
                    tbuf[j, pl.ds(16, 16)] = plsc.load_gather(bufa, [ri1, jv])
                    tbuf[CH + j, pl.ds(0, 16)] = plsc.load_gather(bufb, [ri0, jv])
                    tbuf[CH + j, pl.ds(16, 16)] = plsc.load_gather(bufb, [ri1, jv])

                pltpu.sync_copy(tbuf, tmf_out.at[cc, pl.ds(b * TMD, TMD)])
    return k


def _tcu_body(tmf_ref, e_ref, u_ref):
    u_ref[...] = jnp.dot(tmf_ref[0] + tmf_ref[1], e_ref[...],
                         preferred_element_type=_f32)


def _tcu(tmf, e32):
    blk = 2048
    return pl.pallas_call(
        _tcu_body,
        grid=(B * TMD // blk,),
        in_specs=[pl.BlockSpec((2, blk, 32), lambda i: (0, i, 0)),
                  pl.BlockSpec((32, 128), lambda i: (0, 0))],
        out_specs=pl.BlockSpec((blk, 128), lambda i: (i, 0)),
        out_shape=jax.ShapeDtypeStruct((B * TMD, 128), _f32),
    )(tmf, e32)


def _tch_body(gmaxp, gsump, x3, w1f, bc1, wx1, bx1, out2f, wx2, bx2, bc2rep,
              wfcg1, bfcg1, wfcg2, bfcg2, wfc1, bfc1, wfc2, bfc2, wout, bout,
              out_ref):
    sums = [jnp.sum(gsump[cidx], axis=0) for cidx in range(NCHK)]
    cnt = sums[NCHK - 1][:, CH - 1:CH]
    inv = 1.0 / jnp.maximum(cnt, 1.0)
    parts = [jnp.max(gmaxp[cidx], axis=0) for cidx in range(NCHK)]
    parts += [sm * inv for sm in sums]
    g = jnp.concatenate(parts, axis=1)
    g = jnp.maximum(jnp.dot(g, wfcg1[...], preferred_element_type=_f32)
                    + bfcg1[...], 0.0)
    g = jnp.dot(g, wfcg2[...], preferred_element_type=_f32) + bfcg2[...]

    xt1 = bx1[...]
    for p in range(17):
        o1p = jnp.dot(x3[p], w1f[...], preferred_element_type=_f32) + bc1[...]
        xt1 = xt1 + jnp.dot(o1p, wx1[p * 32:(p + 1) * 32, :],
                            preferred_element_type=_f32)

    xt2 = (jnp.dot(out2f[...], wx2[...], preferred_element_type=_f32)
           + jnp.dot(bc2rep[...], wx2[...], preferred_element_type=_f32)
           + bx2[...])

    xc = jnp.concatenate([g, xt1, xt2], axis=1)
    h = jnp.maximum(jnp.dot(xc, wfc1[...], preferred_element_type=_f32)
                    + bfc1[...], 0.0)
    h = jnp.maximum(jnp.dot(h, wfc2[...], preferred_element_type=_f32)
                    + bfc2[...], 0.0)
    out_ref[...] = jnp.dot(h, wout[...], preferred_element_type=_f32) + bout[...]


def _tch(*args):
    return pl.pallas_call(
        _tch_body,
        out_shape=jax.ShapeDtypeStruct((B, 1), _f32),
    )(*args)


def kernel(x, target1, W_gat, a_src, a_dst, b_gat, W_gcn, b_gcn, W_fcg1, b_fcg1,
           W_fcg2, b_fcg2, emb_xt, w_c2, b_c2, W_fc2xt, b_fc2xt, w_c1, b_c1,
           W_fc1xt, b_fc1xt, W_fc1, b_fc1, W_fc2, b_fc2, W_out, b_out,
           edge_index, batch, target2):
    x_pad = jnp.pad(x, ((0, NP - N), (0, 0)))
    wg_pad = jnp.zeros((F0, HFP), _f32).at[:, _COLMAP].set(W_gat)
    asm = jnp.zeros((HFP, 16), _f32).at[_COLMAP, _HEADCOL].set(a_src.reshape(-1))
    adm = jnp.zeros((HFP, 16), _f32).at[_COLMAP, _HEADCOL].set(a_dst.reshape(-1))
    bg_pad = jnp.zeros((1, HFP), _f32).at[0, _COLMAP].set(b_gat)
    wgcn_pad = (jnp.zeros((HFP, HFP), _f32)
                .at[np.ix_(_COLMAP, _COLMAP)].set(W_gcn))
    bgcn_pad = jnp.zeros((1, HFP), _f32).at[0, _COLMAP].set(b_gcn)
    wfcg1_pad = (jnp.zeros((2 * HFP, 1500), _f32)
                 .at[np.concatenate([_COLMAP, HFP + _COLMAP])].set(W_fcg1))

    loops = jnp.arange(N, dtype=_i32)
    srcp = jnp.concatenate(
        [edge_index[0].astype(_i32), loops,
         jnp.zeros((EPAD - N - E,), _i32)]).reshape(NWORK, NBLK, KE)
    dstp = jnp.concatenate(
        [edge_index[1].astype(_i32), loops,
         jnp.full((EPAD - N - E,), DUMMY, _i32)]).reshape(NWORK, NBLK, KE)
    batch_pad = jnp.concatenate(
        [batch.astype(_i32), jnp.full((NP - N,), B + 2, _i32)])

    zero16 = jnp.zeros((NP, 16), _f32)
    zero128 = jnp.zeros((NP, CH), _f32)

    t2idx = (jnp.arange(B, dtype=_i32)[:, None] * 32 + target2.astype(_i32))
    t2idx = jnp.pad(t2idx.reshape(NWORK, 4000), ((0, 0), (0, 96)),
                    constant_values=4096).reshape(NWORK, 32, KE)
    widx = jnp.pad(jnp.tile(jnp.arange(1000, dtype=_i32), 4),
                   (0, 96)).reshape(32, KE)
    w2d = jnp.transpose(w_c2, (2, 0, 1)).reshape(TMD, 1000).T
    w2da, w2db = w2d[:, :CH], w2d[:, CH:]
    e32 = jnp.pad(emb_xt, ((0, 6), (0, 0)))

    x3 = jnp.stack([target1[:, :, p:p + 8].reshape(B, 160) for p in range(17)])
    wx1 = W_fc1xt.reshape(32, 17, 128).transpose(1, 0, 2).reshape(544, 128)
    w1f = jnp.transpose(w_c1, (1, 2, 0)).reshape(160, 32)
    bc2rep = jnp.repeat(b_c2, 121)[None, :]

    outs = _tca(x_pad, wg_pad, asm, adm)
    xwc, es16, ed16 = outs[:NCHK], outs[NCHK], outs[NCHK + 1]
    den_parts = _scb()(es16, ed16, srcp, dstp, zero16)
    comb = _tcc(den_parts, es16, ed16)
    an = _scb2()(comb, srcp, dstp)
    gat_parts = _sc_aggregate(True)(*xwc, srcp, dstp, an, zero128)
    xw2c = _tce(gat_parts, bg_pad, wgcn_pad)
    gcn_parts = _sc_aggregate(False)(*xw2c, srcp, dstp, an, zero128)
    h2c = _tcg1(gcn_parts, bgcn_pad)
    gmaxp, gsump = _scg()(*h2c, batch_pad, zero128)

    tm_parts = _sch2()(w2da, w2db, widx, t2idx, zero128)
    tmf = _sct()(tm_parts)
    u = _tcu(tmf, e32)

    u3 = u.reshape(B, TMD, 128)
    out2 = sum(u3[:, 32 * kk:32 * kk + 32, kk:kk + 121] for kk in range(8))
    out2f = out2.reshape(B, 32 * 121)

    return _tch(gmaxp, gsump, x3, w1f, b_c1[None, :], wx1, b_fc1xt[None, :],
                out2f, W_fc2xt, b_fc2xt[None, :], bc2rep,
                wfcg1_pad, b_fcg1[None, :], W_fcg2, b_fcg2[None, :],
                W_fc1, b_fc1[None, :], W_fc2, b_fc2[None, :],
                W_out, b_out[None, :])


# speedup vs baseline: 6.5113x; 1.0388x over previous
import functools

import jax
import jax.numpy as jnp
import numpy as np
from jax import lax
from jax.experimental import pallas as pl
from jax.experimental.pallas import tpu as pltpu
from jax.experimental.pallas import tpu_sc as plsc

N = 10000
E = 160000
B = 128
F0 = 78
H = 10
FP = 80
CH = 128
NCHK = 7
HFP = NCHK * CH
NP = 10240
DUMMY = 10100
NWORK = 32
EPW = 5376
NBLK = 42
KE = 128
EPAD = NWORK * EPW

TM_ROWS = 4128
TMD = 256

_COLMAP = np.concatenate([np.arange(h * FP, h * FP + F0) for h in range(H)])
_HEADCOL = np.repeat(np.arange(H), F0)
_VREG_HEAD = [[(c * CH + 16 * j) // FP for j in range(8)] for c in range(NCHK)]

_f32 = jnp.float32
_i32 = jnp.int32


def _mesh():
    return plsc.VectorSubcoreMesh(core_axis_name="c", subcore_axis_name="s",
                                  num_cores=2, num_subcores=16)


def _tca_body(x_ref, wg_ref, asm_ref, adm_ref, *outs):
    xwp = jnp.dot(x_ref[...], wg_ref[...], preferred_element_type=_f32)
    for cidx in range(NCHK):
        outs[cidx][...] = xwp[:, cidx * CH:(cidx + 1) * CH]
    outs[NCHK][...] = jnp.dot(xwp, asm_ref[...], preferred_element_type=_f32)
    outs[NCHK + 1][...] = jnp.dot(xwp, adm_ref[...], preferred_element_type=_f32)


def _tca(x_pad, wg_pad, asm, adm):
    blk = 512
    outs = [jax.ShapeDtypeStruct((NP, CH), _f32) for _ in range(NCHK)]
    outs += [jax.ShapeDtypeStruct((NP, 16), _f32)] * 2
    return pl.pallas_call(
        _tca_body,
        grid=(NP // blk,),
        in_specs=[pl.BlockSpec((blk, F0), lambda i: (i, 0)),
                  pl.BlockSpec((F0, HFP), lambda i: (0, 0)),
                  pl.BlockSpec((HFP, 16), lambda i: (0, 0)),
                  pl.BlockSpec((HFP, 16), lambda i: (0, 0))],
        out_specs=[pl.BlockSpec((blk, CH), lambda i: (i, 0))] * NCHK
                  + [pl.BlockSpec((blk, 16), lambda i: (i, 0))] * 2,
        out_shape=outs,
    )(x_pad, wg_pad, asm, adm)


def _scb():
    @functools.partial(
        pl.kernel,
        out_type=jax.ShapeDtypeStruct((2, NP, 16), _f32),
        mesh=_mesh(),
        compiler_params=pltpu.CompilerParams(use_tc_tiling_on_sc=False),
        scratch_types=[pltpu.VMEM((NBLK, KE), _i32),
                       pltpu.VMEM((NBLK, KE), _i32),
                       pltpu.VMEM((KE, 16), _f32),
                       pltpu.VMEM((KE, 16), _f32),
                       pltpu.VMEM((KE, 16), _f32),
                       pltpu.VMEM_SHARED((NP, 16), _f32),
                       pltpu.SemaphoreType.DMA],
    )
    def k(es_hbm, ed_hbm, srcp, dstp, zero16, den_parts,
          srcv, dstv, esr, edr, eeb, acc, sem):
        c = lax.axis_index("c")
        s = lax.axis_index("s")
        w = s * 2 + c
        lane = lax.iota(_i32, 16)
        keep = lane < 10
        one10 = jnp.where(lane == 10, 1.0, 0.0).astype(_f32)

        @pl.when(s == 0)
        def _():
            pltpu.sync_copy(zero16, acc)
        plsc.subcore_barrier()

        pltpu.sync_copy(srcp.at[w], srcv)
        pltpu.sync_copy(dstp.at[w], dstv)

        @pl.loop(0, NBLK)
        def _blk(blk):
            pltpu.async_copy(es_hbm.at[srcv.at[blk]], esr, sem).wait()
            pltpu.async_copy(ed_hbm.at[dstv.at[blk]], edr, sem).wait()

            @pl.loop(0, KE)
            def _edge(i):
                v = esr[i, :] + edr[i, :]
                v = jnp.maximum(v, 0.0) + 0.2 * jnp.minimum(v, 0.0)
                ee = jnp.exp(v)
                eeb[i, :] = jnp.where(keep, ee, one10)

            pltpu.sync_copy(eeb, acc.at[dstv.at[blk]], add=True)

        plsc.subcore_barrier()
        nr = NP // 16
        pltpu.sync_copy(acc.at[pl.ds(s * nr, nr)],
                        den_parts.at[c, pl.ds(s * nr, nr)])
    return k


def _tcc_body(dp_ref, es_ref, ed_ref, out_ref):
    d = dp_ref[0] + dp_ref[1]
    deg = d[:, 10:11]
    dinv = jnp.where(deg > 0, lax.rsqrt(jnp.maximum(deg, 1e-30)), 0.0)
    col = lax.broadcasted_iota(_i32, (NP, 16), 1)
    denb = jnp.where(col == 10, dinv, d + 1e-16)
    out_ref[...] = jnp.concatenate([es_ref[...], ed_ref[...], denb], axis=1)


def _tcc(den_parts, es16, ed16):
    return pl.pallas_call(
        _tcc_body,
        out_shape=jax.ShapeDtypeStruct((NP, 48), _f32),
    )(den_parts, es16, ed16)


def _scb2():
    @functools.partial(
        pl.kernel,
        out_type=jax.ShapeDtypeStruct((NWORK, EPW, 16), _f32),
        mesh=_mesh(),
        compiler_params=pltpu.CompilerParams(use_tc_tiling_on_sc=False),
        scratch_types=[pltpu.VMEM((NBLK, KE), _i32),
                       pltpu.VMEM((NBLK, KE), _i32),
                       pltpu.VMEM((KE, 48), _f32),
                       pltpu.VMEM((KE, 48), _f32),
                       pltpu.VMEM((KE, 16), _f32),
                       pltpu.SemaphoreType.DMA],
    )
    def k(ct_hbm, srcp, dstp, an_out, srcv, dstv, rs, rd, anb, sem):
        c = lax.axis_index("c")
        s = lax.axis_index("s")
        w = s * 2 + c
        lane = lax.iota(_i32, 16)
        keep = lane < 10
        one10 = jnp.where(lane == 10, 1.0, 0.0).astype(_f32)
        is10 = lane == 10

        pltpu.sync_copy(srcp.at[w], srcv)
        pltpu.sync_copy(dstp.at[w], dstv)

        @pl.loop(0, NBLK)
        def _blk(blk):
            pltpu.async_copy(ct_hbm.at[srcv.at[blk]], rs, sem).wait()
            pltpu.async_copy(ct_hbm.at[dstv.at[blk]], rd, sem).wait()

            @pl.loop(0, KE)
            def _edge(i):
                v = rs[i, pl.ds(0, 16)] + rd[i, pl.ds(16, 16)]
                v = jnp.maximum(v, 0.0) + 0.2 * jnp.minimum(v, 0.0)
                ee = jnp.where(keep, jnp.exp(v), one10)
                dfs = rs[i, pl.ds(32, 16)]
                dfd = rd[i, pl.ds(32, 16)]
                a = jnp.where(keep, ee / dfd, 0.0)
                anb[i, :] = jnp.where(is10, dfs * dfd, a)

            pltpu.sync_copy(anb, an_out.at[w, pl.ds(blk * KE, KE)])
    return k


def _sc_aggregate(gat):
    @functools.partial(
        pl.kernel,
        out_type=jax.ShapeDtypeStruct((NCHK, 2, NP, CH), _f32),
        mesh=_mesh(),
        compiler_params=pltpu.CompilerParams(use_tc_tiling_on_sc=False),
        scratch_types=[pltpu.VMEM((NBLK, KE), _i32),
                       pltpu.VMEM((NBLK, KE), _i32),
                       pltpu.VMEM((KE, 16), _f32),
                       pltpu.VMEM((KE, 16), _f32),
                       pltpu.VMEM((KE, CH), _f32),
                       pltpu.VMEM((KE, CH), _f32),
                       pltpu.VMEM_SHARED((NP, CH), _f32),
                       pltpu.SemaphoreType.DMA,
                       pltpu.SemaphoreType.DMA],
    )
    def k(xw0, xw1, xw2, xw3, xw4, xw5, xw6, srcp, dstp, an_hbm, zero128, out,
          srcv, dstv, anb0, anb1, rows0, rows1, acc, sem, sem2):
        c = lax.axis_index("c")
        s = lax.axis_index("s")
        w = s * 2 + c

        pltpu.sync_copy(srcp.at[w], srcv)
        pltpu.sync_copy(dstp.at[w], dstv)

        for cidx, xw in enumerate((xw0, xw1, xw2, xw3, xw4, xw5, xw6)):
            heads = _VREG_HEAD[cidx]

            @pl.when(s == 0)
            def _():
                pltpu.sync_copy(zero128, acc)
            plsc.subcore_barrier()

            pltpu.async_copy(xw.at[srcv.at[0]], rows0, sem)
            if gat:
                pltpu.async_copy(an_hbm.at[w, pl.ds(0, KE)], anb0, sem2)

            @pl.loop(0, NBLK // 2)
            def _blk2(b2):
                for ph, (rows, anb, rnxt, anxt) in enumerate(
                        ((rows0, anb0, rows1, anb1),
                         (rows1, anb1, rows0, anb0))):
                    blk = b2 * 2 + ph
                    pltpu.make_async_copy(xw.at[srcv.at[blk]], rows, sem).wait()
                    if gat:
                        pltpu.make_async_copy(
                            an_hbm.at[w, pl.ds(blk * KE, KE)], anb, sem2).wait()

                    @pl.when(blk + 1 < NBLK)
                    def _():
                        pltpu.async_copy(xw.at[srcv.at[blk + 1]], rnxt, sem)
                        if gat:
                            pltpu.async_copy(
                                an_hbm.at[w, pl.ds((blk + 1) * KE, KE)], anxt,
                                sem2)

                    if gat:
                        @pl.loop(0, KE, unroll=4)
                        def _edge(i):
                            av = anb[i, :]
                            hs = sorted(set(heads))
                            sv = {h: jnp.full((16,), av[h], _f32) for h in hs}
                            scale = [sv[h] for h in heads]
                            for j in range(8):
                                rows[i, pl.ds(16 * j, 16)] = (
                                    rows[i, pl.ds(16 * j, 16)] * scale[j])

                    pltpu.sync_copy(rows, acc.at[dstv.at[blk]], add=True)

            plsc.subcore_barrier()
            nr = NP // 16
            pltpu.sync_copy(acc.at[pl.ds(s * nr, nr)],
                            out.at[cidx, c, pl.ds(s * nr, nr)])
            plsc.subcore_barrier()
    return k


def _tce_body(gp_ref, bg_ref, wg_ref, cb_ref, *outs):
    parts = [jnp.maximum(gp_ref[c, 0] + gp_ref[c, 1]
                         + bg_ref[0, c * CH:(c + 1) * CH], 0.0)
             for c in range(NCHK)]
    h1 = jnp.concatenate(parts, axis=1)
    dinv = cb_ref[:, 42:43]
    xw2 = jnp.dot(h1, wg_ref[...], preferred_element_type=_f32) * dinv
    for cidx in range(NCHK):
        outs[cidx][...] = xw2[:, cidx * CH:(cidx + 1) * CH]


def _tce(gat_parts, bg_pad, wgcn_pad, comb):
    blk = 512
    return pl.pallas_call(
        _tce_body,
        grid=(NP // blk,),
        in_specs=[pl.BlockSpec((NCHK, 2, blk, CH), lambda i: (0, 0, i, 0)),
                  pl.BlockSpec((1, HFP), lambda i: (0, 0)),
                  pl.BlockSpec((HFP, HFP), lambda i: (0, 0)),
                  pl.BlockSpec((blk, 48), lambda i: (i, 0))],
        out_specs=[pl.BlockSpec((blk, CH), lambda i: (i, 0))] * NCHK,
        out_shape=[jax.ShapeDtypeStruct((NP, CH), _f32) for _ in range(NCHK)],
    )(gat_parts, bg_pad, wgcn_pad, comb)


def _tcg1_body(gp_ref, bg_ref, cb_ref, *outs):
    dinv = cb_ref[:, 42:43]
    for cidx in range(NCHK):
        outs[cidx][...] = jnp.maximum(
            (gp_ref[cidx, 0] + gp_ref[cidx, 1]) * dinv
            + bg_ref[0, cidx * CH:(cidx + 1) * CH], 0.0)


def _tcg1(gcn_parts, bg_pad, comb):
    blk = 512
    return pl.pallas_call(
        _tcg1_body,
        grid=(NP // blk,),
        in_specs=[pl.BlockSpec((NCHK, 2, blk, CH), lambda i: (0, 0, i, 0)),
                  pl.BlockSpec((1, HFP), lambda i: (0, 0)),
                  pl.BlockSpec((blk, 48), lambda i: (i, 0))],
        out_specs=[pl.BlockSpec((blk, CH), lambda i: (i, 0))] * NCHK,
        out_shape=[jax.ShapeDtypeStruct((NP, CH), _f32) for _ in range(NCHK)],
    )(gcn_parts, bg_pad, comb)


def _scg():
    rpw = NP // NWORK
    @functools.partial(
        pl.kernel,
        out_type=[jax.ShapeDtypeStruct((NCHK, NWORK, B, CH), _f32),
                  jax.ShapeDtypeStruct((NCHK, NWORK, B, CH), _f32)],
        mesh=_mesh(),
        compiler_params=pltpu.CompilerParams(use_tc_tiling_on_sc=False),
        scratch_types=[pltpu.VMEM((rpw,), _i32),
                       pltpu.VMEM((64, CH), _f32),
                       pltpu.VMEM((B, CH), _f32),
                       pltpu.VMEM((B, CH), _f32),
                       pltpu.SemaphoreType.DMA],
    )
    def k(h0, h1, h2, h3, h4, h5, h6, batch_hbm, zero128, gmaxp, gsump,
          bv, rowb, pmax, psum, sem):
        c = lax.axis_index("c")
        s = lax.axis_index("s")
        w = s * 2 + c
        one15 = jnp.where(lax.iota(_i32, 16) == 15, 1.0, 0.0).astype(_f32)

        pltpu.sync_copy(batch_hbm.at[pl.ds(w * rpw, rpw)], bv)

        for cidx, h2c in enumerate((h0, h1, h2, h3, h4, h5, h6)):
            pltpu.sync_copy(zero128.at[pl.ds(0, B)], pmax)
            pltpu.sync_copy(zero128.at[pl.ds(0, B)], psum)

            @pl.loop(0, rpw // 64)
            def _sb(sb):
                pltpu.sync_copy(h2c.at[pl.ds(w * rpw + sb * 64, 64)], rowb)

                @pl.loop(0, 4)
                def _grp(gi):
                    idv = bv[pl.ds(sb * 64 + gi * 16, 16)]
                    for jj in range(16):
                        g = idv[jj]
                        i = gi * 16 + jj

                        @pl.when(g < B)
                        def _():
                            for j in range(8):
                                r = rowb[i, pl.ds(16 * j, 16)]
                                pmax[g, pl.ds(16 * j, 16)] = jnp.maximum(
                                    pmax[g, pl.ds(16 * j, 16)], r)
                                psum[g, pl.ds(16 * j, 16)] = (
                                    psum[g, pl.ds(16 * j, 16)] + r)
                            if cidx == NCHK - 1:
                                psum[g, pl.ds(112, 16)] = (
                                    psum[g, pl.ds(112, 16)] + one15)

            pltpu.sync_copy(pmax, gmaxp.at[cidx, w])
            pltpu.sync_copy(psum, gsump.at[cidx, w])
    return k


def _sch2():
    nblk = 32
    @functools.partial(
        pl.kernel,
        out_type=jax.ShapeDtypeStruct((2, 2, TM_ROWS, CH), _f32),
        mesh=_mesh(),
        compiler_params=pltpu.CompilerParams(use_tc_tiling_on_sc=False),
        scratch_types=[pltpu.VMEM((nblk, KE), _i32),
                       pltpu.VMEM((nblk, KE), _i32),
                       pltpu.VMEM((KE, CH), _f32),
                       pltpu.VMEM((KE, CH), _f32),
                       pltpu.VMEM_SHARED((TM_ROWS, CH), _f32),
                       pltpu.VMEM_SHARED((TM_ROWS, CH), _f32),
                       pltpu.SemaphoreType.DMA],
    )
    def k(w2da, w2db, widx, t2idx, zero128, tm_parts,
          wiv, tiv, rowsa, rowsb, acca, accb, sem):
        c = lax.axis_index("c")
        s = lax.axis_index("s")
        w = s * 2 + c

        @pl.when(s == 0)
        def _():
            pltpu.sync_copy(zero128.at[pl.ds(0, TM_ROWS)], acca)
            pltpu.sync_copy(zero128.at[pl.ds(0, TM_ROWS)], accb)
        plsc.subcore_barrier()

        pltpu.sync_copy(widx, wiv)
        pltpu.sync_copy(t2idx.at[w], tiv)

        @pl.loop(0, nblk)
        def _blk(blk):
            pltpu.async_copy(w2da.at[wiv.at[blk]], rowsa, sem).wait()
            pltpu.sync_copy(rowsa, acca.at[tiv.at[blk]], add=True)
            pltpu.async_copy(w2db.at[wiv.at[blk]], rowsb, sem).wait()
            pltpu.sync_copy(rowsb, accb.at[tiv.at[blk]], add=True)

        plsc.subcore_barrier()
        nr = TM_ROWS // 16
        pltpu.sync_copy(acca.at[pl.ds(s * nr, nr)],
                        tm_parts.at[c, 0, pl.ds(s * nr, nr)])
        pltpu.sync_copy(accb.at[pl.ds(s * nr, nr)],
                        tm_parts.at[c, 1, pl.ds(s * nr, nr)])
    return k


def _sct():
    bpw = B // NWORK
    @functools.partial(
        pl.kernel,
        out_type=jax.ShapeDtypeStruct((2, B * TMD, 32), _f32),
        mesh=_mesh(),
        compiler_params=pltpu.CompilerParams(use_tc_tiling_on_sc=False,
                                             needs_layout_passes=False),
        scratch_types=[pltpu.VMEM((32, CH), _f32),
                       pltpu.VMEM((32, CH), _f32),
                       pltpu.VMEM((TMD, 32), _f32),
                       pltpu.SemaphoreType.DMA],
    )
    def k(tm_parts, tmf_out, bufa, bufb, tbuf, sem):
        c = lax.axis_index("c")
        s = lax.axis_index("s")
        w = s * 2 + c

        for cc in range(2):
            @pl.loop(0, bpw)
            def _b(bi):
                b = w * bpw + bi
                pltpu.sync_copy(tm_parts.at[cc, 0, pl.ds(b * 32, 32)], bufa)
                pltpu.sync_copy(tm_parts.at[cc, 1, pl.ds(b * 32, 32)], bufb)

                @pl.loop(0, CH)
                def _j(j):
                    jv = jnp.full((16,), j, _i32)
                    ri0 = lax.iota(_i32, 16)
                    ri1 = ri0 + 16
                    tbuf[j, pl.ds(0, 16)] = plsc.load_gather(bufa, [ri0, jv])
                    tbuf[j, pl.ds(16, 16)] = plsc.load_gather(bufa, [ri1, jv])
                    tbuf[CH + j, pl.ds(0, 16)] = plsc.load_gather(bufb, [ri0, jv])
                    tbuf[CH + j, pl.ds(16, 16)] = plsc.load_gather(bufb, [ri1, jv])

                pltpu.sync_copy(tbuf, tmf_out.at[cc, pl.ds(b * TMD, TMD)])
    return k


def _tcu_body(tmf_ref, e_ref, u_ref):
    u_ref[...] = jnp.dot(tmf_ref[0] + tmf_ref[1], e_ref[...],
                         preferred_element_type=_f32)


def _tcu(tmf, e32):
    blk = 2048
    return pl.pallas_call(
        _tcu_body,
        grid=(B * TMD // blk,),
        in_specs=[pl.BlockSpec((2, blk, 32), lambda i: (0, i, 0)),
                  pl.BlockSpec((32, 128), lambda i: (0, 0))],
        out_specs=pl.BlockSpec((blk, 128), lambda i: (i, 0)),
        out_shape=jax.ShapeDtypeStruct((B * TMD, 128), _f32),
    )(tmf, e32)


def _tch_body(gmaxp, gsump, x3, w1f, bc1, wx1, bx1, out2f, wx2, bx2, bc2rep,
              wfcg1, bfcg1, wfcg2, bfcg2, wfc1, bfc1, wfc2, bfc2, wout, bout,
              out_ref):
    sums = [jnp.sum(gsump[cidx], axis=0) for cidx in range(NCHK)]
    cnt = sums[NCHK - 1][:, CH - 1:CH]
    inv = 1.0 / jnp.maximum(cnt, 1.0)
    parts = [jnp.max(gmaxp[cidx], axis=0) for cidx in range(NCHK)]
    parts += [sm * inv for sm in sums]
    g = jnp.concatenate(parts, axis=1)
    g = jnp.maximum(jnp.dot(g, wfcg1[...], preferred_element_type=_f32)
                    + bfcg1[...], 0.0)
    g = jnp.dot(g, wfcg2[...], preferred_element_type=_f32) + bfcg2[...]

    xt1 = bx1[...]
    for p in range(17):
        o1p = jnp.dot(x3[p], w1f[...], preferred_element_type=_f32) + bc1[...]
        xt1 = xt1 + jnp.dot(o1p, wx1[p * 32:(p + 1) * 32, :],
                            preferred_element_type=_f32)

    xt2 = (jnp.dot(out2f[...], wx2[...], preferred_element_type=_f32)
           + jnp.dot(bc2rep[...], wx2[...], preferred_element_type=_f32)
           + bx2[...])

    xc = jnp.concatenate([g, xt1, xt2], axis=1)
    h = jnp.maximum(jnp.dot(xc, wfc1[...], preferred_element_type=_f32)
                    + bfc1[...], 0.0)
    h = jnp.maximum(jnp.dot(h, wfc2[...], preferred_element_type=_f32)
                    + bfc2[...], 0.0)
    out_ref[...] = jnp.dot(h, wout[...], preferred_element_type=_f32) + bout[...]


def _tch(*args):
    return pl.pallas_call(
        _tch_body,
        out_shape=jax.ShapeDtypeStruct((B, 1), _f32),
    )(*args)


def kernel(x, target1, W_gat, a_src, a_dst, b_gat, W_gcn, b_gcn, W_fcg1, b_fcg1,
           W_fcg2, b_fcg2, emb_xt, w_c2, b_c2, W_fc2xt, b_fc2xt, w_c1, b_c1,
           W_fc1xt, b_fc1xt, W_fc1, b_fc1, W_fc2, b_fc2, W_out, b_out,
           edge_index, batch, target2):
    x_pad = jnp.pad(x, ((0, NP - N), (0, 0)))
    wg_pad = jnp.zeros((F0, HFP), _f32).at[:, _COLMAP].set(W_gat)
    asm = jnp.zeros((HFP, 16), _f32).at[_COLMAP, _HEADCOL].set(a_src.reshape(-1))
    adm = jnp.zeros((HFP, 16), _f32).at[_COLMAP, _HEADCOL].set(a_dst.reshape(-1))
    bg_pad = jnp.zeros((1, HFP), _f32).at[0, _COLMAP].set(b_gat)
    wgcn_pad = (jnp.zeros((HFP, HFP), _f32)
                .at[np.ix_(_COLMAP, _COLMAP)].set(W_gcn))
    bgcn_pad = jnp.zeros((1, HFP), _f32).at[0, _COLMAP].set(b_gcn)
    wfcg1_pad = (jnp.zeros((2 * HFP, 1500), _f32)
                 .at[np.concatenate([_COLMAP, HFP + _COLMAP])].set(W_fcg1))

    loops = jnp.arange(N, dtype=_i32)
    srcp = jnp.concatenate(
        [edge_index[0].astype(_i32), loops,
         jnp.zeros((EPAD - N - E,), _i32)]).reshape(NWORK, NBLK, KE)
    dstp = jnp.concatenate(
        [edge_index[1].astype(_i32), loops,
         jnp.full((EPAD - N - E,), DUMMY, _i32)]).reshape(NWORK, NBLK, KE)
    batch_pad = jnp.concatenate(
        [batch.astype(_i32), jnp.full((NP - N,), B + 2, _i32)])

    zero16 = jnp.zeros((NP, 16), _f32)
    zero128 = jnp.zeros((NP, CH), _f32)

    t2idx = (jnp.arange(B, dtype=_i32)[:, None] * 32 + target2.astype(_i32))
    t2idx = jnp.pad(t2idx.reshape(NWORK, 4000), ((0, 0), (0, 96)),
                    constant_values=4096).reshape(NWORK, 32, KE)
    widx = jnp.pad(jnp.tile(jnp.arange(1000, dtype=_i32), 4),
                   (0, 96)).reshape(32, KE)
    w2d = jnp.transpose(w_c2, (2, 0, 1)).reshape(TMD, 1000).T
    w2da, w2db = w2d[:, :CH], w2d[:, CH:]
    e32 = jnp.pad(emb_xt, ((0, 6), (0, 0)))

    x3 = jnp.stack([target1[:, :, p:p + 8].reshape(B, 160) for p in range(17)])
    wx1 = W_fc1xt.reshape(32, 17, 128).transpose(1, 0, 2).reshape(544, 128)
    w1f = jnp.transpose(w_c1, (1, 2, 0)).reshape(160, 32)
    bc2rep = jnp.repeat(b_c2, 121)[None, :]

    outs = _tca(x_pad, wg_pad, asm, adm)
    xwc, es16, ed16 = outs[:NCHK], outs[NCHK], outs[NCHK + 1]
    den_parts = _scb()(es16, ed16, srcp, dstp, zero16)
    comb = _tcc(den_parts, es16, ed16)
    an = _scb2()(comb, srcp, dstp)
    gat_parts = _sc_aggregate(True)(*xwc, srcp, dstp, an, zero128)
    xw2c = _tce(gat_parts, bg_pad, wgcn_pad, comb)
    gcn_parts = _sc_aggregate(False)(*xw2c, srcp, dstp, an, zero128)
    h2c = _tcg1(gcn_parts, bgcn_pad, comb)
    gmaxp, gsump = _scg()(*h2c, batch_pad, zero128)

    tm_parts = _sch2()(w2da, w2db, widx, t2idx, zero128)
    tmf = _sct()(tm_parts)
    u = _tcu(tmf, e32)

    u3 = u.reshape(B, TMD, 128)
    out2 = sum(u3[:, 32 * kk:32 * kk + 32, kk:kk + 121] for kk in range(8))
    out2f = out2.reshape(B, 32 * 121)

    return _tch(gmaxp, gsump, x3, w1f, b_c1[None, :], wx1, b_fc1xt[None, :],
                out2f, W_fc2xt, b_fc2xt[None, :], bc2rep,
                wfcg1_pad, b_fcg1[None, :], W_fcg2, b_fcg2[None, :],
                W_fc1, b_fc1[None, :], W_fc2, b_fc2[None, :],
                W_out, b_out[None, :])
